# baseline (speedup 1.0000x reference)
import jax
import jax.numpy as jnp
from jax.experimental import pallas as pl
from jax.experimental.pallas import tpu as pltpu

HD = 128
N_BIG = 100000
N_STR = 10000
BLK = 16384
RPB = BLK // HD
NB_GRID = -(-N_BIG // BLK)
STR_BLKS = -(-N_STR // BLK)
ROWS = NB_GRID * RPB
NSTRIPE = ROWS // 8
K = 32
NEG_INF = float("-inf")
BIG = 2**31 - 1


def _router_kernel(query_ref, w_ref, b_ref, sem_ref, ctx_ref, str_ref,
                   sem_vals_ref, sem_idx_ref, ctx_vals_ref, ctx_idx_ref,
                   str_out, q_scr, s_sem, s_ctx, pm_sem, pm_ctx):
    i = pl.program_id(0)

    @pl.when(i == 0)
    def _init():
        q = jax.lax.dot_general(
            w_ref[...], query_ref[...], (((1,), (0,)), ((), ())),
            preferred_element_type=jnp.float32)
        q_scr[...] = q + b_ref[...]

    q = q_scr[...]

    idx16 = (jax.lax.broadcasted_iota(jnp.int32, (RPB, HD), 0) * HD
             + jax.lax.broadcasted_iota(jnp.int32, (RPB, HD), 1))
    valid = (i * BLK + idx16) < N_BIG

    sem_sim = jax.lax.dot_general(
        sem_ref[...], q, (((1,), (0,)), ((), ())),
        preferred_element_type=jnp.float32).reshape(RPB, HD)
    s_sem[pl.ds(i * RPB, RPB), :] = jnp.where(valid, sem_sim, NEG_INF)

    ctx_sim = jax.lax.dot_general(
        ctx_ref[...], q, (((1,), (0,)), ((), ())),
        preferred_element_type=jnp.float32).reshape(RPB, HD)
    s_ctx[pl.ds(i * RPB, RPB), :] = jnp.where(valid, ctx_sim, NEG_INF)

    @pl.when(i < STR_BLKS)
    def _str():
        str_out[...] = jax.lax.dot_general(
            str_ref[...], q, (((1,), (0,)), ((), ())),
            preferred_element_type=jnp.float32).reshape(RPB, HD)

    @pl.when(i == NB_GRID - 1)
    def _extract():
        pm_sem[...] = jnp.max(s_sem[...].reshape(NSTRIPE, 8, HD), axis=1)
        pm_ctx[...] = jnp.max(s_ctx[...].reshape(NSTRIPE, 8, HD), axis=1)

        stripe_iota = jax.lax.broadcasted_iota(jnp.int32, (NSTRIPE, HD), 0)
        idx8 = (jax.lax.broadcasted_iota(jnp.int32, (8, HD), 0) * HD
                + jax.lax.broadcasted_iota(jnp.int32, (8, HD), 1))
        kidx = jax.lax.broadcasted_iota(jnp.int32, (1, K), 1)

        def extract_one(s_scr, pm_scr):
            pm = pm_scr[...]
            m = jnp.max(pm)
            smin = jnp.min(jnp.where(pm == m, stripe_iota, BIG))
            v = s_scr[pl.ds(smin * 8, 8), :]
            am = jnp.min(jnp.where(v == m, idx8, BIG))
            v2 = jnp.where(idx8 == am, NEG_INF, v)
            s_scr[pl.ds(smin * 8, 8), :] = v2
            pm_scr[pl.ds(smin, 1), :] = jnp.max(v2, axis=0, keepdims=True)
            return m, smin * (8 * HD) + am

        def body(j, carry):
            sv, si, cv, ci = carry
            ms, gs = extract_one(s_sem, pm_sem)
            mc, gc = extract_one(s_ctx, pm_ctx)
            sel = kidx == j
            return (jnp.where(sel, ms, sv), jnp.where(sel, gs, si),
                    jnp.where(sel, mc, cv), jnp.where(sel, gc, ci))

        z = jnp.zeros((1, K), jnp.float32)
        zi = jnp.zeros((1, K), jnp.int32)
        sv, si, cv, ci = jax.lax.fori_loop(0, K, body, (z, zi, z, zi))
        sem_vals_ref[...] = sv
        sem_idx_ref[...] = si
        ctx_vals_ref[...] = cv
        ctx_idx_ref[...] = ci


def _gather_kernel(idx_ref, sem_ref, ctx_ref, out_ref):
    i = pl.program_id(0)
    out_ref[...] = jnp.where(i < K, sem_ref[...], ctx_ref[...])


@jax.jit
def _run(query_embedding, semantic, structural, contextual, W, b):
    q_col = query_embedding.reshape(HD, 1)
    b_col = b.reshape(HD, 1)

    sem_vals, sem_idx, ctx_vals, ctx_idx, str_sims = pl.pallas_call(
        _router_kernel,
        grid=(NB_GRID,),
        in_specs=[
            pl.BlockSpec((HD, 1), lambda i: (0, 0)),
            pl.BlockSpec((HD, HD), lambda i: (0, 0)),
            pl.BlockSpec((HD, 1), lambda i: (0, 0)),
            pl.BlockSpec((BLK, HD), lambda i: (i, 0)),
            pl.BlockSpec((BLK, HD), lambda i: (i, 0)),
            pl.BlockSpec((BLK, HD), lambda i: (jnp.minimum(i, STR_BLKS - 1), 0)),
        ],
        out_specs=[
            pl.BlockSpec((1, K), lambda i: (0, 0)),
            pl.BlockSpec((1, K), lambda i: (0, 0)),
            pl.BlockSpec((1, K), lambda i: (0, 0)),
            pl.BlockSpec((1, K), lambda i: (0, 0)),
            pl.BlockSpec((RPB, HD), lambda i: (jnp.minimum(i, STR_BLKS - 1), 0)),
        ],
        out_shape=[
            jax.ShapeDtypeStruct((1, K), jnp.float32),
            jax.ShapeDtypeStruct((1, K), jnp.int32),
            jax.ShapeDtypeStruct((1, K), jnp.float32),
            jax.ShapeDtypeStruct((1, K), jnp.int32),
            jax.ShapeDtypeStruct((STR_BLKS * RPB, HD), jnp.float32),
        ],
        scratch_shapes=[
            pltpu.VMEM((HD, 1), jnp.float32),
            pltpu.VMEM((ROWS, HD), jnp.float32),
            pltpu.VMEM((ROWS, HD), jnp.float32),
            pltpu.VMEM((NSTRIPE, HD), jnp.float32),
            pltpu.VMEM((NSTRIPE, HD), jnp.float32),
        ],
    )(q_col, W, b_col, semantic, contextual, structural)

    all_idx = jnp.concatenate([sem_idx[0], ctx_idx[0]])
    grid_spec = pltpu.PrefetchScalarGridSpec(
        num_scalar_prefetch=1,
        grid=(2 * K,),
        in_specs=[
            pl.BlockSpec((1, 1, HD), lambda i, idx: (idx[i], 0, 0)),
            pl.BlockSpec((1, 1, HD), lambda i, idx: (idx[i], 0, 0)),
        ],
        out_specs=pl.BlockSpec((1, 1, HD), lambda i, idx: (i, 0, 0)),
    )
    sel = pl.pallas_call(
        _gather_kernel,
        grid_spec=grid_spec,
        out_shape=jax.ShapeDtypeStruct((2 * K, 1, HD), jnp.float32),
    )(all_idx, semantic.reshape(-1, 1, HD), contextual.reshape(-1, 1, HD))
    sel = sel.reshape(2 * K, HD)

    all_weights = jnp.concatenate(
        [sem_vals[0], str_sims.reshape(-1)[:N_STR], ctx_vals[0]])
    return sel[:K], structural, sel[K:], all_weights


def kernel(query_embedding, semantic, structural, contextual, W, b, top_k):
    return _run(query_embedding, semantic, structural, contextual, W, b)


# speedup vs baseline: 1.1218x; 1.1218x over previous
import functools

import jax
import jax.numpy as jnp
from jax import lax
from jax.experimental import pallas as pl
from jax.experimental.pallas import tpu as pltpu
from jax.experimental.pallas import tpu_sc as plsc

HD = 128
N_BIG = 100000
N_STR = 10000
BLK = 8192
RPB = BLK // HD
NB_GRID = -(-N_BIG // BLK)
STR_BLKS = -(-N_STR // BLK)
ROWS = NB_GRID * RPB
NSTRIPE = ROWS // 8
K = 32
NEG_INF = float("-inf")
BIG = 2**31 - 1


def _router_kernel(query_ref, w_ref, b_ref, sem_ref, ctx_ref, str_ref,
                   sem_vals_ref, sem_idx_ref, ctx_vals_ref, ctx_idx_ref,
                   str_out, q_scr, s_sem, s_ctx, pm_sem, pm_ctx):
    i = pl.program_id(0)

    @pl.when(i == 0)
    def _init():
        q = jax.lax.dot_general(
            w_ref[...], query_ref[...], (((1,), (0,)), ((), ())),
            preferred_element_type=jnp.float32)
        q_scr[...] = q + b_ref[...]

    q = q_scr[...]

    idx16 = (jax.lax.broadcasted_iota(jnp.int32, (RPB, HD), 0) * HD
             + jax.lax.broadcasted_iota(jnp.int32, (RPB, HD), 1))
    valid = (i * BLK + idx16) < N_BIG

    sem_sim = jax.lax.dot_general(
        sem_ref[...], q, (((1,), (0,)), ((), ())),
        preferred_element_type=jnp.float32).reshape(RPB, HD)
    s_sem[pl.ds(i * RPB, RPB), :] = jnp.where(valid, sem_sim, NEG_INF)

    ctx_sim = jax.lax.dot_general(
        ctx_ref[...], q, (((1,), (0,)), ((), ())),
        preferred_element_type=jnp.float32).reshape(RPB, HD)
    s_ctx[pl.ds(i * RPB, RPB), :] = jnp.where(valid, ctx_sim, NEG_INF)

    @pl.when(i < STR_BLKS)
    def _str():
        str_out[...] = jax.lax.dot_general(
            str_ref[...], q, (((1,), (0,)), ((), ())),
            preferred_element_type=jnp.float32).reshape(RPB, HD)

    @pl.when(i == NB_GRID - 1)
    def _extract():
        pm_sem[...] = jnp.max(s_sem[...].reshape(NSTRIPE, 8, HD), axis=1)
        pm_ctx[...] = jnp.max(s_ctx[...].reshape(NSTRIPE, 8, HD), axis=1)

        stripe_iota = jax.lax.broadcasted_iota(jnp.int32, (NSTRIPE, HD), 0)
        idx8 = (jax.lax.broadcasted_iota(jnp.int32, (8, HD), 0) * HD
                + jax.lax.broadcasted_iota(jnp.int32, (8, HD), 1))
        kidx = jax.lax.broadcasted_iota(jnp.int32, (1, K), 1)

        def extract_one(s_scr, pm_scr):
            pm = pm_scr[...]
            m = jnp.max(pm)
            smin = jnp.min(jnp.where(pm == m, stripe_iota, BIG))
            v = s_scr[pl.ds(smin * 8, 8), :]
            am = jnp.min(jnp.where(v == m, idx8, BIG))
            v2 = jnp.where(idx8 == am, NEG_INF, v)
            s_scr[pl.ds(smin * 8, 8), :] = v2
            pm_scr[pl.ds(smin, 1), :] = jnp.max(v2, axis=0, keepdims=True)
            return m, smin * (8 * HD) + am

        def body(j, carry):
            sv, si, cv, ci = carry
            ms, gs = extract_one(s_sem, pm_sem)
            mc, gc = extract_one(s_ctx, pm_ctx)
            sel = kidx == j
            return (jnp.where(sel, ms, sv), jnp.where(sel, gs, si),
                    jnp.where(sel, mc, cv), jnp.where(sel, gc, ci))

        z = jnp.zeros((1, K), jnp.float32)
        zi = jnp.zeros((1, K), jnp.int32)
        sv, si, cv, ci = jax.lax.fori_loop(0, K, body, (z, zi, z, zi))
        sem_vals_ref[...] = sv
        sem_idx_ref[...] = si
        ctx_vals_ref[...] = cv
        ctx_idx_ref[...] = ci


def _sc_gather(idx_hbm, sem_hbm, ctx_hbm, out_hbm, idx_v, rows_v, dsem):
    wid = lax.axis_index("s") * 2 + lax.axis_index("c")

    @pl.when(wid < 4)
    def _():
        base = pl.multiple_of(wid * 16, 16)
        pltpu.sync_copy(idx_hbm.at[pl.ds(base, 16)], idx_v)

        @pl.when(wid < 2)
        def _():
            pltpu.async_copy(sem_hbm.at[idx_v], rows_v, dsem).wait()

        @pl.when(wid >= 2)
        def _():
            pltpu.async_copy(ctx_hbm.at[idx_v], rows_v, dsem).wait()

        pltpu.sync_copy(rows_v, out_hbm.at[pl.ds(base, 16)])


@jax.jit
def _run(query_embedding, semantic, structural, contextual, W, b):
    q_col = query_embedding.reshape(HD, 1)
    b_col = b.reshape(HD, 1)

    sem_vals, sem_idx, ctx_vals, ctx_idx, str_sims = pl.pallas_call(
        _router_kernel,
        grid=(NB_GRID,),
        in_specs=[
            pl.BlockSpec((HD, 1), lambda i: (0, 0)),
            pl.BlockSpec((HD, HD), lambda i: (0, 0)),
            pl.BlockSpec((HD, 1), lambda i: (0, 0)),
            pl.BlockSpec((BLK, HD), lambda i: (i, 0)),
            pl.BlockSpec((BLK, HD), lambda i: (i, 0)),
            pl.BlockSpec((BLK, HD), lambda i: (jnp.minimum(i, STR_BLKS - 1), 0)),
        ],
        out_specs=[
            pl.BlockSpec((1, K), lambda i: (0, 0)),
            pl.BlockSpec((1, K), lambda i: (0, 0)),
            pl.BlockSpec((1, K), lambda i: (0, 0)),
            pl.BlockSpec((1, K), lambda i: (0, 0)),
            pl.BlockSpec((RPB, HD), lambda i: (jnp.minimum(i, STR_BLKS - 1), 0)),
        ],
        out_shape=[
            jax.ShapeDtypeStruct((1, K), jnp.float32),
            jax.ShapeDtypeStruct((1, K), jnp.int32),
            jax.ShapeDtypeStruct((1, K), jnp.float32),
            jax.ShapeDtypeStruct((1, K), jnp.int32),
            jax.ShapeDtypeStruct((STR_BLKS * RPB, HD), jnp.float32),
        ],
        scratch_shapes=[
            pltpu.VMEM((HD, 1), jnp.float32),
            pltpu.VMEM((ROWS, HD), jnp.float32),
            pltpu.VMEM((ROWS, HD), jnp.float32),
            pltpu.VMEM((NSTRIPE, HD), jnp.float32),
            pltpu.VMEM((NSTRIPE, HD), jnp.float32),
        ],
    )(q_col, W, b_col, semantic, contextual, structural)

    all_idx = jnp.concatenate([sem_idx[0], ctx_idx[0]])
    sel = pl.kernel(
        _sc_gather,
        mesh=plsc.VectorSubcoreMesh(core_axis_name="c", subcore_axis_name="s"),
        out_type=jax.ShapeDtypeStruct((2 * K, HD), jnp.float32),
        scratch_types=[
            pltpu.VMEM((16,), jnp.int32),
            pltpu.VMEM((16, HD), jnp.float32),
            pltpu.SemaphoreType.DMA,
        ],
    )(all_idx, semantic, contextual)

    all_weights = jnp.concatenate(
        [sem_vals[0], str_sims.reshape(-1)[:N_STR], ctx_vals[0]])
    return sel[:K], structural, sel[K:], all_weights


def kernel(query_embedding, semantic, structural, contextual, W, b, top_k):
    return _run(query_embedding, semantic, structural, contextual, W, b)


# speedup vs baseline: 1.1624x; 1.0362x over previous
import jax
import jax.numpy as jnp
from jax import lax
from jax.experimental import pallas as pl
from jax.experimental.pallas import tpu as pltpu
from jax.experimental.pallas import tpu_sc as plsc

HD = 128
N_BIG = 100000
N_STR = 10000
BLK = 10240
RPB = BLK // HD
NB_GRID = -(-N_BIG // BLK)
STR_BLKS = -(-N_STR // BLK)
ROWS = NB_GRID * RPB
NSTRIPE = ROWS // 8
K = 32
NEG_INF = float("-inf")
BIG = 2**31 - 1


def _router_kernel(query_ref, w_ref, b_ref, sem_ref, ctx_ref, str_ref,
                   sem_vals_ref, sem_idx_ref, ctx_vals_ref, ctx_idx_ref,
                   str_out, q_scr, s_sem, s_ctx, pm_sem, pm_ctx):
    i = pl.program_id(0)

    @pl.when(i == 0)
    def _init():
        q = jax.lax.dot_general(
            w_ref[...], query_ref[...], (((1,), (0,)), ((), ())),
            preferred_element_type=jnp.float32)
        q_scr[...] = q + b_ref[...]

    q = q_scr[...]

    idx16 = (jax.lax.broadcasted_iota(jnp.int32, (RPB, HD), 0) * HD
             + jax.lax.broadcasted_iota(jnp.int32, (RPB, HD), 1))
    valid = (i * BLK + idx16) < N_BIG

    sem_sim = jax.lax.dot_general(
        sem_ref[...], q, (((1,), (0,)), ((), ())),
        preferred_element_type=jnp.float32).reshape(RPB, HD)
    s_sem[pl.ds(i * RPB, RPB), :] = jnp.where(valid, sem_sim, NEG_INF)

    ctx_sim = jax.lax.dot_general(
        ctx_ref[...], q, (((1,), (0,)), ((), ())),
        preferred_element_type=jnp.float32).reshape(RPB, HD)
    s_ctx[pl.ds(i * RPB, RPB), :] = jnp.where(valid, ctx_sim, NEG_INF)

    @pl.when(i < STR_BLKS)
    def _str():
        str_out[...] = jax.lax.dot_general(
            str_ref[...], q, (((1,), (0,)), ((), ())),
            preferred_element_type=jnp.float32).reshape(RPB, HD)

    @pl.when(i == NB_GRID - 1)
    def _extract():
        pm_sem[...] = jnp.max(s_sem[...].reshape(NSTRIPE, 8, HD), axis=1)
        pm_ctx[...] = jnp.max(s_ctx[...].reshape(NSTRIPE, 8, HD), axis=1)

        stripe_iota = jax.lax.broadcasted_iota(jnp.int32, (NSTRIPE, HD), 0)
        idx8 = (jax.lax.broadcasted_iota(jnp.int32, (8, HD), 0) * HD
                + jax.lax.broadcasted_iota(jnp.int32, (8, HD), 1))
        kidx = jax.lax.broadcasted_iota(jnp.int32, (1, K), 1)

        def extract_one(s_scr, pm_scr):
            pm = pm_scr[...]
            m = jnp.max(pm)
            smin = jnp.min(jnp.where(pm == m, stripe_iota, BIG))
            v = s_scr[pl.ds(smin * 8, 8), :]
            am = jnp.min(jnp.where(v == m, idx8, BIG))
            v2 = jnp.where(idx8 == am, NEG_INF, v)
            s_scr[pl.ds(smin * 8, 8), :] = v2
            pm_scr[pl.ds(smin, 1), :] = jnp.max(v2, axis=0, keepdims=True)
            return m, smin * (8 * HD) + am

        def body(j, carry):
            sv, si, cv, ci = carry
            ms, gs = extract_one(s_sem, pm_sem)
            mc, gc = extract_one(s_ctx, pm_ctx)
            sel = kidx == j
            return (jnp.where(sel, ms, sv), jnp.where(sel, gs, si),
                    jnp.where(sel, mc, cv), jnp.where(sel, gc, ci))

        z = jnp.zeros((1, K), jnp.float32)
        zi = jnp.zeros((1, K), jnp.int32)
        sv, si, cv, ci = jax.lax.fori_loop(0, K, body, (z, zi, z, zi))
        sem_vals_ref[...] = sv
        sem_idx_ref[...] = si
        ctx_vals_ref[...] = cv
        ctx_idx_ref[...] = ci


def _sc_gather(idx_hbm, sem_hbm, ctx_hbm, out_hbm, idx_v, rows_v, dsem):
    wid = lax.axis_index("s") * 2 + lax.axis_index("c")

    @pl.when(wid < 4)
    def _():
        base = pl.multiple_of(wid * 16, 16)
        pltpu.sync_copy(idx_hbm.at[pl.ds(base, 16)], idx_v)

        @pl.when(wid < 2)
        def _():
            pltpu.async_copy(sem_hbm.at[idx_v], rows_v, dsem).wait()

        @pl.when(wid >= 2)
        def _():
            pltpu.async_copy(ctx_hbm.at[idx_v], rows_v, dsem).wait()

        pltpu.sync_copy(rows_v, out_hbm.at[pl.ds(base, 16)])


@jax.jit
def _run(query_embedding, semantic, structural, contextual, W, b):
    q_col = query_embedding.reshape(HD, 1)
    b_col = b.reshape(HD, 1)

    sem_vals, sem_idx, ctx_vals, ctx_idx, str_sims = pl.pallas_call(
        _router_kernel,
        grid=(NB_GRID,),
        in_specs=[
            pl.BlockSpec((HD, 1), lambda i: (0, 0)),
            pl.BlockSpec((HD, HD), lambda i: (0, 0)),
            pl.BlockSpec((HD, 1), lambda i: (0, 0)),
            pl.BlockSpec((BLK, HD), lambda i: (i, 0)),
            pl.BlockSpec((BLK, HD), lambda i: (i, 0)),
            pl.BlockSpec((BLK, HD), lambda i: (jnp.minimum(i, STR_BLKS - 1), 0)),
        ],
        out_specs=[
            pl.BlockSpec((1, K), lambda i: (0, 0)),
            pl.BlockSpec((1, K), lambda i: (0, 0)),
            pl.BlockSpec((1, K), lambda i: (0, 0)),
            pl.BlockSpec((1, K), lambda i: (0, 0)),
            pl.BlockSpec((RPB, HD), lambda i: (jnp.minimum(i, STR_BLKS - 1), 0)),
        ],
        out_shape=[
            jax.ShapeDtypeStruct((1, K), jnp.float32),
            jax.ShapeDtypeStruct((1, K), jnp.int32),
            jax.ShapeDtypeStruct((1, K), jnp.float32),
            jax.ShapeDtypeStruct((1, K), jnp.int32),
            jax.ShapeDtypeStruct((STR_BLKS * RPB, HD), jnp.float32),
        ],
        scratch_shapes=[
            pltpu.VMEM((HD, 1), jnp.float32),
            pltpu.VMEM((ROWS, HD), jnp.float32),
            pltpu.VMEM((ROWS, HD), jnp.float32),
            pltpu.VMEM((NSTRIPE, HD), jnp.float32),
            pltpu.VMEM((NSTRIPE, HD), jnp.float32),
        ],
    )(q_col, W, b_col, semantic, contextual, structural)

    all_idx = jnp.concatenate([sem_idx[0], ctx_idx[0]])
    sel = pl.kernel(
        _sc_gather,
        mesh=plsc.VectorSubcoreMesh(core_axis_name="c", subcore_axis_name="s"),
        out_type=jax.ShapeDtypeStruct((2 * K, HD), jnp.float32),
        scratch_types=[
            pltpu.VMEM((16,), jnp.int32),
            pltpu.VMEM((16, HD), jnp.float32),
            pltpu.SemaphoreType.DMA,
        ],
    )(all_idx, semantic, contextual)

    all_weights = jnp.concatenate(
        [sem_vals[0], str_sims.reshape(-1)[:N_STR], ctx_vals[0]])
    return sel[:K], structural, sel[K:], all_weights


def kernel(query_embedding, semantic, structural, contextual, W, b, top_k):
    return _run(query_embedding, semantic, structural, contextual, W, b)


# speedup vs baseline: 1.1632x; 1.0007x over previous
import jax
import jax.numpy as jnp
from jax import lax
from jax.experimental import pallas as pl
from jax.experimental.pallas import tpu as pltpu
from jax.experimental.pallas import tpu_sc as plsc

HD = 128
N_BIG = 100000
N_STR = 10000
BLK = 12800
RPB = BLK // HD
NB_GRID = -(-N_BIG // BLK)
STR_BLKS = -(-N_STR // BLK)
ROWS = NB_GRID * RPB
NSTRIPE = ROWS // 8
K = 32
NEG_INF = float("-inf")
BIG = 2**31 - 1


def _router_kernel(query_ref, w_ref, b_ref, sem_ref, ctx_ref, str_ref,
                   sem_vals_ref, sem_idx_ref, ctx_vals_ref, ctx_idx_ref,
                   str_out, q_scr, s_sem, s_ctx, pm_sem, pm_ctx):
    i = pl.program_id(0)

    @pl.when(i == 0)
    def _init():
        q = jax.lax.dot_general(
            w_ref[...], query_ref[...], (((1,), (0,)), ((), ())),
            preferred_element_type=jnp.float32)
        q_scr[...] = q + b_ref[...]

    q = q_scr[...]

    idx16 = (jax.lax.broadcasted_iota(jnp.int32, (RPB, HD), 0) * HD
             + jax.lax.broadcasted_iota(jnp.int32, (RPB, HD), 1))
    valid = (i * BLK + idx16) < N_BIG

    sem_sim = jax.lax.dot_general(
        sem_ref[...], q, (((1,), (0,)), ((), ())),
        preferred_element_type=jnp.float32).reshape(RPB, HD)
    s_sem[pl.ds(i * RPB, RPB), :] = jnp.where(valid, sem_sim, NEG_INF)

    ctx_sim = jax.lax.dot_general(
        ctx_ref[...], q, (((1,), (0,)), ((), ())),
        preferred_element_type=jnp.float32).reshape(RPB, HD)
    s_ctx[pl.ds(i * RPB, RPB), :] = jnp.where(valid, ctx_sim, NEG_INF)

    @pl.when(i < STR_BLKS)
    def _str():
        str_out[...] = jax.lax.dot_general(
            str_ref[...], q, (((1,), (0,)), ((), ())),
            preferred_element_type=jnp.float32).reshape(RPB, HD)

    @pl.when(i == NB_GRID - 1)
    def _extract():
        pm_sem[...] = jnp.max(s_sem[...].reshape(NSTRIPE, 8, HD), axis=1)
        pm_ctx[...] = jnp.max(s_ctx[...].reshape(NSTRIPE, 8, HD), axis=1)

        stripe_iota = jax.lax.broadcasted_iota(jnp.int32, (NSTRIPE, HD), 0)
        idx8 = (jax.lax.broadcasted_iota(jnp.int32, (8, HD), 0) * HD
                + jax.lax.broadcasted_iota(jnp.int32, (8, HD), 1))
        kidx = jax.lax.broadcasted_iota(jnp.int32, (1, K), 1)

        def extract_one(s_scr, pm_scr):
            pm = pm_scr[...]
            m = jnp.max(pm)
            smin = jnp.min(jnp.where(pm == m, stripe_iota, BIG))
            v = s_scr[pl.ds(smin * 8, 8), :]
            am = jnp.min(jnp.where(v == m, idx8, BIG))
            v2 = jnp.where(idx8 == am, NEG_INF, v)
            s_scr[pl.ds(smin * 8, 8), :] = v2
            pm_scr[pl.ds(smin, 1), :] = jnp.max(v2, axis=0, keepdims=True)
            return m, smin * (8 * HD) + am

        def body(j, carry):
            sv, si, cv, ci = carry
            ms, gs = extract_one(s_sem, pm_sem)
            mc, gc = extract_one(s_ctx, pm_ctx)
            sel = kidx == j
            return (jnp.where(sel, ms, sv), jnp.where(sel, gs, si),
                    jnp.where(sel, mc, cv), jnp.where(sel, gc, ci))

        z = jnp.zeros((1, K), jnp.float32)
        zi = jnp.zeros((1, K), jnp.int32)
        sv, si, cv, ci = jax.lax.fori_loop(0, K, body, (z, zi, z, zi))
        sem_vals_ref[...] = sv
        sem_idx_ref[...] = si
        ctx_vals_ref[...] = cv
        ctx_idx_ref[...] = ci


def _sc_gather(idx_hbm, sem_hbm, ctx_hbm, out_hbm, idx_v, rows_v, dsem):
    wid = lax.axis_index("s") * 2 + lax.axis_index("c")

    @pl.when(wid < 4)
    def _():
        base = pl.multiple_of(wid * 16, 16)
        pltpu.sync_copy(idx_hbm.at[pl.ds(base, 16)], idx_v)

        @pl.when(wid < 2)
        def _():
            pltpu.async_copy(sem_hbm.at[idx_v], rows_v, dsem).wait()

        @pl.when(wid >= 2)
        def _():
            pltpu.async_copy(ctx_hbm.at[idx_v], rows_v, dsem).wait()

        pltpu.sync_copy(rows_v, out_hbm.at[pl.ds(base, 16)])


@jax.jit
def _run(query_embedding, semantic, structural, contextual, W, b):
    q_col = query_embedding.reshape(HD, 1)
    b_col = b.reshape(HD, 1)

    sem_vals, sem_idx, ctx_vals, ctx_idx, str_sims = pl.pallas_call(
        _router_kernel,
        grid=(NB_GRID,),
        in_specs=[
            pl.BlockSpec((HD, 1), lambda i: (0, 0)),
            pl.BlockSpec((HD, HD), lambda i: (0, 0)),
            pl.BlockSpec((HD, 1), lambda i: (0, 0)),
            pl.BlockSpec((BLK, HD), lambda i: (i, 0)),
            pl.BlockSpec((BLK, HD), lambda i: (i, 0)),
            pl.BlockSpec((BLK, HD), lambda i: (jnp.minimum(i, STR_BLKS - 1), 0)),
        ],
        out_specs=[
            pl.BlockSpec((1, K), lambda i: (0, 0)),
            pl.BlockSpec((1, K), lambda i: (0, 0)),
            pl.BlockSpec((1, K), lambda i: (0, 0)),
            pl.BlockSpec((1, K), lambda i: (0, 0)),
            pl.BlockSpec((RPB, HD), lambda i: (jnp.minimum(i, STR_BLKS - 1), 0)),
        ],
        out_shape=[
            jax.ShapeDtypeStruct((1, K), jnp.float32),
            jax.ShapeDtypeStruct((1, K), jnp.int32),
            jax.ShapeDtypeStruct((1, K), jnp.float32),
            jax.ShapeDtypeStruct((1, K), jnp.int32),
            jax.ShapeDtypeStruct((STR_BLKS * RPB, HD), jnp.float32),
        ],
        scratch_shapes=[
            pltpu.VMEM((HD, 1), jnp.float32),
            pltpu.VMEM((ROWS, HD), jnp.float32),
            pltpu.VMEM((ROWS, HD), jnp.float32),
            pltpu.VMEM((NSTRIPE, HD), jnp.float32),
            pltpu.VMEM((NSTRIPE, HD), jnp.float32),
        ],
    )(q_col, W, b_col, semantic, contextual, structural)

    all_idx = jnp.concatenate([sem_idx[0], ctx_idx[0]])
    sel = pl.kernel(
        _sc_gather,
        mesh=plsc.VectorSubcoreMesh(core_axis_name="c", subcore_axis_name="s"),
        out_type=jax.ShapeDtypeStruct((2 * K, HD), jnp.float32),
        scratch_types=[
            pltpu.VMEM((16,), jnp.int32),
            pltpu.VMEM((16, HD), jnp.float32),
            pltpu.SemaphoreType.DMA,
        ],
    )(all_idx, semantic, contextual)

    all_weights = jnp.concatenate(
        [sem_vals[0], str_sims.reshape(-1)[:N_STR], ctx_vals[0]])
    return sel[:K], structural, sel[K:], all_weights


def kernel(query_embedding, semantic, structural, contextual, W, b, top_k):
    return _run(query_embedding, semantic, structural, contextual, W, b)
